# baseline (speedup 1.0000x reference)
import functools

import jax
import jax.numpy as jnp
from jax import lax
from jax.experimental import pallas as pl
from jax.experimental.pallas import tpu as pltpu
from jax.experimental.pallas import tpu_sc as plsc

N = 10000
E = 320000
D = 128

NC = 2
NS = 16
NW = NC * NS
CHUNK = 128
NBLK = E // CHUNK
NCB = NBLK // NW
NXB = NBLK - NCB * NW
NPAD = 10240
RPT = NPAD // NS


def _prep_body(x_ref, wh_ref, w_ref):
    x = x_ref[...]
    a = lax.dot_general(x, wh_ref[...], (((1,), (1,)), ((), ())),
                        preferred_element_type=jnp.float32)
    scores = jnp.sum(a * x, axis=1, keepdims=True) / jnp.sqrt(jnp.float32(D))
    m = jnp.max(scores)
    e = jnp.exp(scores - m)
    attn = e / jnp.sum(e)
    w_ref[...] = x * attn


_prep = pl.pallas_call(
    _prep_body,
    out_shape=jax.ShapeDtypeStruct((N, D), jnp.float32),
)


_sc_mesh = plsc.VectorSubcoreMesh(core_axis_name="c", subcore_axis_name="s")


NBUF = 2


@functools.partial(
    pl.kernel,
    out_type=jax.ShapeDtypeStruct((NC, NPAD, D), jnp.float32),
    mesh=_sc_mesh,
    scratch_types=[
        [pltpu.VMEM((2, CHUNK), jnp.int32)] * NBUF,
        pltpu.VMEM((NBUF, CHUNK, D), jnp.float32),
        pltpu.VMEM_SHARED((NPAD, D), jnp.float32),
        [pltpu.SemaphoreType.DMA] * NBUF,
        [pltpu.SemaphoreType.DMA] * NBUF,
    ],
)
def _sc_scatter(edge_hbm, w_hbm, out_hbm,
                ebufs, rows_v, acc_sh, esems, gsems):
    c = lax.axis_index("c")
    s = lax.axis_index("s")
    wid = s * NC + c
    blk0 = wid * NCB

    def start_efetch(block, q):
        off = pl.multiple_of(block * CHUNK, CHUNK)
        return pltpu.async_copy(edge_hbm.at[:, pl.ds(off, CHUNK)],
                                ebufs[q], esems[q])

    def start_gather(q, r):
        return pltpu.async_copy(w_hbm.at[ebufs[q].at[0]], rows_v.at[r],
                                gsems[r])

    def scat(q, r):
        pltpu.sync_copy(rows_v.at[r], acc_sh.at[ebufs[q].at[1]], add=True)

    ew = [start_efetch(blk0 + q, q) for q in range(NBUF)]

    zero16 = jnp.zeros((16,), jnp.float32)

    def zfill(i, _):
        for j in range(D // 16):
            rows_v[0, i, pl.ds(j * 16, 16)] = zero16
        return ()

    lax.fori_loop(0, CHUNK, zfill, ())
    for k in range(RPT // CHUNK):
        pltpu.sync_copy(rows_v.at[0],
                        acc_sh.at[pl.ds(s * RPT + k * CHUNK, CHUNK)])

    ew[0].wait()
    gw0 = start_gather(0, 0)
    plsc.subcore_barrier()

    def wait_gather(r):
        pltpu.make_async_copy(w_hbm.at[ebufs[r].at[0]], rows_v.at[r],
                              gsems[r]).wait()

    def body(t, _):
        ch = t * NBUF
        ew[1].wait()
        g1 = start_gather(1, 1)
        gw0.wait()
        scat(0, 0)
        start_efetch(blk0 + ch + 2, 0)
        ew[0].wait()
        start_gather(0, 0)
        g1.wait()
        scat(1, 1)
        start_efetch(blk0 + ch + 3, 1)
        return ()

    npair = (NCB - NBUF) // NBUF
    lax.fori_loop(0, npair, body, ())
    ew[1].wait()
    start_gather(1, 1)
    gw0.wait()
    scat(0, 0)
    start_efetch(NW * NCB + lax.rem(wid, NXB), 0)
    ew[0].wait()
    wait_gather(1)
    scat(1, 1)

    @pl.when(wid < NXB)
    def _():
        start_gather(0, 0).wait()
        scat(0, 0)

    plsc.subcore_barrier()
    pltpu.sync_copy(acc_sh.at[pl.ds(s * RPT, RPT)],
                    out_hbm.at[c, pl.ds(s * RPT, RPT)])


def _out_body(p_ref, wo_ref, o_ref):
    h = p_ref[0, :N] + p_ref[1, :N]
    o = jnp.dot(h, wo_ref[...], preferred_element_type=jnp.float32)
    o_ref[...] = jnp.maximum(o, 0.0)


_finish = pl.pallas_call(
    _out_body,
    out_shape=jax.ShapeDtypeStruct((N, D), jnp.float32),
)


def kernel(x, edge_index, W_head, W_out):
    w = _prep(x, W_head)
    parts = _sc_scatter(edge_index, w)
    return _finish(parts, W_out)


# speedup vs baseline: 1.1181x; 1.1181x over previous
import functools

import jax
import jax.numpy as jnp
from jax import lax
from jax.experimental import pallas as pl
from jax.experimental.pallas import tpu as pltpu
from jax.experimental.pallas import tpu_sc as plsc

N = 10000
E = 320000
D = 128

NC = 2
NS = 16
NW = NC * NS
CHUNK = 128
NBLK = E // CHUNK
NCB = NBLK // NW
NXB = NBLK - NCB * NW
NPAD = 10240
RPT = NPAD // NS


def _prep_body(x_ref, wh_ref, w_ref):
    x = x_ref[...]
    a = lax.dot_general(x, wh_ref[...], (((1,), (1,)), ((), ())),
                        preferred_element_type=jnp.float32)
    scores = jnp.sum(a * x, axis=1, keepdims=True) / jnp.sqrt(jnp.float32(D))
    m = jnp.max(scores)
    e = jnp.exp(scores - m)
    attn = e / jnp.sum(e)
    w_ref[...] = x * attn


_prep = pl.pallas_call(
    _prep_body,
    out_shape=jax.ShapeDtypeStruct((N, D), jnp.float32),
)


_sc_mesh = plsc.VectorSubcoreMesh(core_axis_name="c", subcore_axis_name="s")


NEB = 4
NRB = 2


@functools.partial(
    pl.kernel,
    out_type=jax.ShapeDtypeStruct((NC, NPAD, D), jnp.float32),
    mesh=_sc_mesh,
    scratch_types=[
        [pltpu.VMEM((2, CHUNK), jnp.int32)] * NEB,
        pltpu.VMEM((NRB, CHUNK, D), jnp.float32),
        pltpu.VMEM_SHARED((NPAD, D), jnp.float32),
        [pltpu.SemaphoreType.DMA] * NEB,
        [pltpu.SemaphoreType.DMA] * NRB,
    ],
)
def _sc_scatter(edge_hbm, w_hbm, out_hbm,
                ebufs, rows_v, acc_sh, esems, gsems):
    c = lax.axis_index("c")
    s = lax.axis_index("s")
    wid = s * NC + c
    blk0 = wid * NCB

    def start_efetch(block, q):
        off = pl.multiple_of(block * CHUNK, CHUNK)
        return pltpu.async_copy(edge_hbm.at[:, pl.ds(off, CHUNK)],
                                ebufs[q], esems[q])

    def start_gather(q, r):
        return pltpu.async_copy(w_hbm.at[ebufs[q].at[0]], rows_v.at[r],
                                gsems[r])

    def wait_gather(r):
        pltpu.make_async_copy(w_hbm.at[ebufs[0].at[0]], rows_v.at[r],
                              gsems[r]).wait()

    def scat(q, r):
        pltpu.sync_copy(rows_v.at[r], acc_sh.at[ebufs[q].at[1]], add=True)

    ew = [start_efetch(blk0 + q, q) for q in range(NEB)]

    zero16 = jnp.zeros((16,), jnp.float32)

    def zfill(i, _):
        for j in range(D // 16):
            rows_v[0, i, pl.ds(j * 16, 16)] = zero16
        return ()

    lax.fori_loop(0, CHUNK, zfill, ())
    for k in range(RPT // CHUNK):
        pltpu.sync_copy(rows_v.at[0],
                        acc_sh.at[pl.ds(s * RPT + k * CHUNK, CHUNK)])

    ew[0].wait()
    start_gather(0, 0)
    ew[1].wait()
    start_gather(1, 1)
    plsc.subcore_barrier()

    def turn(i, cblk, more_gather, refetch):
        wait_gather(i % NRB)
        scat(i % NEB, i % NRB)
        if more_gather:
            ew[(i + 2) % NEB].wait()
            start_gather((i + 2) % NEB, i % NRB)
        if refetch is not None:
            start_efetch(refetch, i % NEB)

    def body(t, _):
        cb = blk0 + t * NEB
        for i in range(NEB):
            turn(i, None, True, cb + i + NEB)
        return ()

    nloop = (NCB - 2 * NEB + 2) // NEB
    lax.fori_loop(0, nloop, body, ())
    xblk = NW * NCB + lax.rem(wid, NXB)
    base = nloop * NEB
    turn(base + 0, None, True, blk0 + base + 4)
    turn(base + 1, None, True, blk0 + base + 5)
    turn(base + 2, None, True, xblk)
    turn(base + 3, None, True, None)
    turn(base + 4, None, False, None)
    ew[(base + 4 + 2) % NEB].wait()
    turn(base + 5, None, False, None)

    @pl.when(wid < NXB)
    def _():
        start_gather((base + 6) % NEB, 0).wait()
        scat((base + 6) % NEB, 0)

    plsc.subcore_barrier()
    pltpu.sync_copy(acc_sh.at[pl.ds(s * RPT, RPT)],
                    out_hbm.at[c, pl.ds(s * RPT, RPT)])


def _out_body(p_ref, wo_ref, o_ref):
    h = p_ref[0, :N] + p_ref[1, :N]
    o = jnp.dot(h, wo_ref[...], preferred_element_type=jnp.float32)
    o_ref[...] = jnp.maximum(o, 0.0)


_finish = pl.pallas_call(
    _out_body,
    out_shape=jax.ShapeDtypeStruct((N, D), jnp.float32),
)


def kernel(x, edge_index, W_head, W_out):
    w = _prep(x, W_head)
    parts = _sc_scatter(edge_index, w)
    return _finish(parts, W_out)


# speedup vs baseline: 1.1794x; 1.0549x over previous
import functools

import jax
import jax.numpy as jnp
from jax import lax
from jax.experimental import pallas as pl
from jax.experimental.pallas import tpu as pltpu
from jax.experimental.pallas import tpu_sc as plsc

N = 10000
E = 320000
D = 128

NC = 2
NS = 16
NW = NC * NS
CHUNK = 128
NBLK = E // CHUNK
NCB = NBLK // NW
NXB = NBLK - NCB * NW
NPAD = 10240
RPT = NPAD // NS


def _prep_body(x_ref, wh_ref, w_ref):
    x = x_ref[...]
    a = lax.dot_general(x, wh_ref[...], (((1,), (1,)), ((), ())),
                        preferred_element_type=jnp.float32)
    scores = jnp.sum(a * x, axis=1, keepdims=True) / jnp.sqrt(jnp.float32(D))
    m = jnp.max(scores)
    e = jnp.exp(scores - m)
    attn = e / jnp.sum(e)
    w_ref[...] = x * attn


_prep = pl.pallas_call(
    _prep_body,
    out_shape=jax.ShapeDtypeStruct((N, D), jnp.float32),
)


_sc_mesh = plsc.VectorSubcoreMesh(core_axis_name="c", subcore_axis_name="s")


NEB = 4
NRB = 4
HC = CHUNK // 2
UPW = NCB * 2


@functools.partial(
    pl.kernel,
    out_type=jax.ShapeDtypeStruct((NC, NPAD, D), jnp.float32),
    mesh=_sc_mesh,
    scratch_types=[
        [pltpu.VMEM((2, CHUNK), jnp.int32)] * NEB,
        pltpu.VMEM((NRB, HC, D), jnp.float32),
        [pltpu.VMEM((HC,), jnp.int32)] * NRB,
        [pltpu.VMEM((HC,), jnp.int32)] * NRB,
        pltpu.VMEM_SHARED((NPAD, D), jnp.float32),
        [pltpu.SemaphoreType.DMA] * NEB,
        [pltpu.SemaphoreType.DMA] * NRB,
        [pltpu.SemaphoreType.DMA] * NRB,
    ],
)
def _sc_scatter(edge_hbm, w_hbm, out_hbm,
                ebufs, rows_v, sclean, dclean, acc_sh, esems, gsems, ssems):
    c = lax.axis_index("c")
    s = lax.axis_index("s")
    wid = s * NC + c
    blk0 = wid * NCB
    xblk = NW * NCB + lax.rem(wid, NXB)

    def start_efetch(block_addr, q):
        off = pl.multiple_of(block_addr * CHUNK, CHUNK)
        pltpu.async_copy(edge_hbm.at[:, pl.ds(off, CHUNK)],
                         ebufs[q], esems[q])

    def wait_e(q):
        pltpu.make_async_copy(edge_hbm.at[:, pl.ds(0, CHUNK)],
                              ebufs[q], esems[q]).wait()

    def copy_ids(q, h, row, bufs, r):
        for j in range(HC // 16):
            bufs[r][pl.ds(j * 16, 16)] = ebufs[q][row, pl.ds(h * HC + j * 16, 16)]

    def start_gather(q, h, r):
        copy_ids(q, h, 0, sclean, r)
        pltpu.async_copy(w_hbm.at[sclean[r]], rows_v.at[r], gsems[r])

    def wait_g(r):
        pltpu.make_async_copy(w_hbm.at[sclean[r]],
                              rows_v.at[r], gsems[r]).wait()

    def copy_dst(q, h, r):
        copy_ids(q, h, 1, dclean, r)

    def scat_async(r):
        pltpu.async_copy(rows_v.at[r], acc_sh.at[dclean[r]], ssems[r],
                         add=True)

    def scat_sync(r):
        pltpu.sync_copy(rows_v.at[r], acc_sh.at[dclean[r]], add=True)

    def wait_s(r):
        pltpu.make_async_copy(rows_v.at[r], acc_sh.at[dclean[r]],
                              ssems[r]).wait()

    for q in range(3):
        start_efetch(blk0 + q, q)
    wait_e(0)
    start_gather(0, 0, 0)
    start_gather(0, 1, 1)
    wait_e(1)
    start_gather(1, 0, 2)

    zero16 = jnp.zeros((16,), jnp.float32)

    def zfill(i, _):
        for j in range(D // 16):
            rows_v[3, i, pl.ds(j * 16, 16)] = zero16
        return ()

    lax.fori_loop(0, HC, zfill, ())
    for k in range(RPT // HC):
        pltpu.sync_copy(rows_v.at[3],
                        acc_sh.at[pl.ds(s * RPT + k * HC, HC)])
    plsc.subcore_barrier()

    def turn(pos, first=False, refetch_addr=None, do_gather=True,
             scat="sync"):
        r, q, h = pos % NRB, (pos // 2) % NEB, pos % 2
        wait_g(r)
        copy_dst(q, h, r)
        if scat == "async":
            scat_async(r)
        elif scat == "sync":
            scat_sync(r)
        if do_gather:
            p3 = pos + 3
            r3, q3, h3 = p3 % NRB, (p3 // 2) % NEB, p3 % 2
            if h3 == 0:
                wait_e(q3)
            if scat == "async" and not first:
                wait_s(r3)
            start_gather(q3, h3, r3)
        if refetch_addr is not None:
            start_efetch(refetch_addr, (pos // 2 + 3) % NEB)

    turn(0, first=True, refetch_addr=blk0 + 3)

    def body(t, _):
        for i in range(8):
            pp = 1 + i
            rf = (blk0 + 4 * t + pp // 2 + 3) if pp % 2 == 0 else None
            turn(pp % 8, refetch_addr=rf)
        return ()

    lax.fori_loop(0, 18, body, ())

    turn(1)
    turn(2, refetch_addr=blk0 + 76)
    turn(3)
    turn(4, refetch_addr=blk0 + 77)
    turn(5)
    turn(6, refetch_addr=xblk)
    turn(7)
    turn(0)
    turn(1)
    turn(2, scat="sync")
    turn(3, do_gather=False, scat="sync")
    wait_g(0)
    copy_dst(2, 0, 0)
    wait_g(1)
    copy_dst(2, 1, 1)

    @pl.when(wid < NXB)
    def _():
        scat_sync(0)
        scat_sync(1)

    plsc.subcore_barrier()
    pltpu.sync_copy(acc_sh.at[pl.ds(s * RPT, RPT)],
                    out_hbm.at[c, pl.ds(s * RPT, RPT)])


def _out_body(p_ref, wo_ref, o_ref):
    h = p_ref[0, :N] + p_ref[1, :N]
    o = jnp.dot(h, wo_ref[...], preferred_element_type=jnp.float32)
    o_ref[...] = jnp.maximum(o, 0.0)


_finish = pl.pallas_call(
    _out_body,
    out_shape=jax.ShapeDtypeStruct((N, D), jnp.float32),
)


def kernel(x, edge_index, W_head, W_out):
    w = _prep(x, W_head)
    parts = _sc_scatter(edge_index, w)
    return _finish(parts, W_out)
